# baseline (speedup 1.0000x reference)
import functools
import math

import jax
import jax.numpy as jnp
from jax import lax
from jax.experimental import pallas as pl
from jax.experimental.pallas import tpu as pltpu
from jax.experimental.pallas import tpu_sc as plsc

D = 128
L = 16
NC = 2
NS = 16
NW = NC * NS
GROUP = 128
NBUF = 3
SCALE = math.sqrt(D)


def _make_sc_lookup(ng):
    mesh = plsc.VectorSubcoreMesh(
        core_axis_name="c", subcore_axis_name="s",
        num_cores=NC, num_subcores=NS)

    @functools.partial(
        pl.kernel,
        out_type=jax.ShapeDtypeStruct((NW, ng, GROUP, D), jnp.float32),
        mesh=mesh,
        scratch_types=[
            pltpu.VMEM((ng, GROUP), jnp.int32),
            pltpu.VMEM((NBUF, GROUP, D), jnp.float32),
            pltpu.VMEM((NBUF, GROUP, D), jnp.float32),
            pltpu.SemaphoreType.DMA,
            pltpu.SemaphoreType.DMA,
        ],
    )
    def body(idx_hbm, table_hbm, out_hbm, idx_v, rows_v, sout_v, gsem, osem):
        wid = lax.axis_index("s") * NC + lax.axis_index("c")
        pltpu.sync_copy(idx_hbm.at[wid], idx_v)

        for b in range(NBUF):
            pltpu.async_copy(table_hbm.at[idx_v.at[b]], rows_v.at[b], gsem)

        def scale_slot(b):
            def row(r, carry):
                for j in range(D // L):
                    sl = pl.ds(j * L, L)
                    sout_v[b, r, sl] = rows_v[b, r, sl] * SCALE
                return carry
            lax.fori_loop(0, GROUP, row, 0, unroll=2)

        def turn(g, b, drain_out, prefetch):
            pltpu.make_async_copy(
                table_hbm.at[idx_v.at[b]], rows_v.at[b], gsem).wait()
            if drain_out:
                pltpu.make_async_copy(
                    sout_v.at[b], out_hbm.at[wid, g], osem).wait()
            scale_slot(b)
            pltpu.async_copy(sout_v.at[b], out_hbm.at[wid, g], osem)
            if prefetch:
                pltpu.async_copy(
                    table_hbm.at[idx_v.at[g + NBUF]], rows_v.at[b], gsem)

        for b in range(NBUF):
            turn(b, b, drain_out=False, prefetch=True)

        steady = (ng - NBUF) // NBUF - 1

        def outer(i, carry):
            t = (i + 1) * NBUF
            for b in range(NBUF):
                turn(t + b, b, drain_out=True, prefetch=True)
            return carry

        lax.fori_loop(0, steady, outer, 0)

        for g in range((steady + 1) * NBUF, ng):
            turn(g, g % NBUF, drain_out=True, prefetch=(g + NBUF < ng))

        for b in range(NBUF):
            pltpu.make_async_copy(
                sout_v.at[b], out_hbm.at[wid, 0], osem).wait()

    return body


def kernel(x, table):
    B, T = x.shape
    n = B * T
    assert n % (NW * GROUP) == 0
    ng = n // (NW * GROUP)
    idx = x.reshape(NW, ng, GROUP)
    if idx.dtype != jnp.int32:
        idx = idx.astype(jnp.int32)
    out = _make_sc_lookup(ng)(idx, table)
    return out.reshape(B, T, D)
